# baseline (speedup 1.0000x reference)
import jax
import jax.numpy as jnp
from jax import lax
from jax.experimental import pallas as pl
from jax.experimental.pallas import tpu as pltpu
from jax.experimental.pallas import tpu_sc as plsc

BS = 2
C = 96
H = W = 384
NPIX = H * W
ACC_PER_TILE = NPIX // 16
ACC_WORDS = NPIX + 16
TRASH = NPIX
PX_PER_TILE = NPIX // 16
GROUPS = PX_PER_TILE // 16
ROWS_PER_TILE = H // 16
CLIP_HI = 385.0

NCH = 4
GR_PER_CH = GROUPS // NCH
CH_PX = GR_PER_CH * 16


def _sc_body(g_hbm, out_hbm, gv,
             i00, i01, i10, i11, w00, w01, w10, w11,
             zbuf, rbuf, acc, sem):
    c = lax.axis_index("c")
    s = lax.axis_index("s")

    gbase = c * (2 * NPIX) + s * PX_PER_TILE
    ld0 = pltpu.make_async_copy(g_hbm.at[pl.ds(gbase, PX_PER_TILE)],
                                gv.at[pl.ds(0, PX_PER_TILE)], sem)
    ld1 = pltpu.make_async_copy(g_hbm.at[pl.ds(gbase + NPIX, PX_PER_TILE)],
                                gv.at[pl.ds(PX_PER_TILE, PX_PER_TILE)], sem)
    ld0.start()
    ld1.start()

    def _zero(k, _):
        zbuf[pl.ds(k * 16, 16)] = jnp.zeros((16,), jnp.float32)
        return _
    lax.fori_loop(0, ACC_PER_TILE // 16, _zero, None)
    pltpu.sync_copy(zbuf, acc.at[pl.ds(s * ACC_PER_TILE, ACC_PER_TILE)])

    plsc.subcore_barrier()
    ld0.wait()
    ld1.wait()

    def _compute(g, _):
        col = g * 16
        gi = gv[pl.ds(g * 16, 16)]
        gj = gv[pl.ds(PX_PER_TILE + g * 16, 16)]
        ti = jnp.minimum(jnp.maximum(gi * 192.0 + 193.0, 0.0), CLIP_HI)
        tj = jnp.minimum(jnp.maximum(gj * 192.0 + 193.0, 0.0), CLIP_HI)
        li = ti.astype(jnp.int32)
        lj = tj.astype(jnp.int32)
        fi = ti - li.astype(jnp.float32)
        fj = tj - lj.astype(jnp.float32)
        base = li * W + lj - (W + 1)
        mi0 = jnp.logical_and(li >= 1, li <= H)
        mi1 = li <= H - 1
        mj0 = jnp.logical_and(lj >= 1, lj <= W)
        mj1 = lj <= W - 1
        i00[pl.ds(col, 16)] = jnp.where(mi0 & mj0, base, TRASH)
        i01[pl.ds(col, 16)] = jnp.where(mi0 & mj1, base + 1, TRASH)
        i10[pl.ds(col, 16)] = jnp.where(mi1 & mj0, base + W, TRASH)
        i11[pl.ds(col, 16)] = jnp.where(mi1 & mj1, base + (W + 1), TRASH)
        a0 = 1.0 - fi
        b0 = 1.0 - fj
        w00[pl.ds(col, 16)] = a0 * b0
        w01[pl.ds(col, 16)] = a0 * fj
        w10[pl.ds(col, 16)] = fi * b0
        w11[pl.ds(col, 16)] = fi * fj
        return _

    descs = []
    for k in range(NCH):
        lax.fori_loop(k * GR_PER_CH, (k + 1) * GR_PER_CH, _compute, None)
        sl = pl.ds(k * CH_PX, CH_PX)
        for iv, wv in ((i00, w00), (i01, w01), (i10, w10), (i11, w11)):
            descs.append(
                pltpu.async_copy(wv.at[sl], acc.at[iv.at[sl]], sem, add=True))
    for d in descs:
        d.wait()

    plsc.subcore_barrier()

    rdescs = []
    for r in range(ROWS_PER_TILE):
        rdescs.append(pltpu.async_copy(
            acc.at[pl.ds(s * ACC_PER_TILE + r * W, W)], rbuf.at[r], sem))
    for d in rdescs:
        d.wait()
    row0 = pl.multiple_of(c * H + s * ROWS_PER_TILE, ROWS_PER_TILE)
    pltpu.sync_copy(rbuf, out_hbm.at[pl.ds(row0, ROWS_PER_TILE)])


_sc_scatter = pl.kernel(
    _sc_body,
    out_type=jax.ShapeDtypeStruct((BS * H, W), jnp.float32),
    mesh=plsc.VectorSubcoreMesh(core_axis_name="c", subcore_axis_name="s"),
    scratch_types=[
        pltpu.VMEM((2 * PX_PER_TILE,), jnp.float32),
        pltpu.VMEM((PX_PER_TILE,), jnp.int32),
        pltpu.VMEM((PX_PER_TILE,), jnp.int32),
        pltpu.VMEM((PX_PER_TILE,), jnp.int32),
        pltpu.VMEM((PX_PER_TILE,), jnp.int32),
        pltpu.VMEM((PX_PER_TILE,), jnp.float32),
        pltpu.VMEM((PX_PER_TILE,), jnp.float32),
        pltpu.VMEM((PX_PER_TILE,), jnp.float32),
        pltpu.VMEM((PX_PER_TILE,), jnp.float32),
        pltpu.VMEM((ACC_PER_TILE,), jnp.float32),
        pltpu.VMEM((ROWS_PER_TILE, W), jnp.float32),
        pltpu.VMEM_SHARED((ACC_WORDS,), jnp.float32),
        pltpu.SemaphoreType.DMA,
    ],
)


CB = 8


def _bcast_body(b_ref, o_ref, s0, s1, sem):
    descs = []
    for b, s in ((0, s0), (1, s1)):
        s[...] = jnp.broadcast_to(b_ref[pl.ds(b * H, H), :][None], (CB, H, W))
        for c in range(0, C, CB):
            d = pltpu.make_async_copy(s, o_ref.at[b, pl.ds(c, CB)], sem)
            d.start()
            descs.append(d)
    for d in descs:
        d.wait()


def kernel(x, inv_grid):
    g = jnp.transpose(inv_grid, (0, 3, 1, 2)).reshape(BS * 2 * NPIX)
    bacc = _sc_scatter(g)

    out = pl.pallas_call(
        _bcast_body,
        in_specs=[pl.BlockSpec(memory_space=pltpu.VMEM)],
        out_specs=pl.BlockSpec(memory_space=pltpu.MemorySpace.HBM),
        out_shape=jax.ShapeDtypeStruct((BS, C, H, W), x.dtype),
        scratch_shapes=[
            pltpu.VMEM((CB, H, W), jnp.float32),
            pltpu.VMEM((CB, H, W), jnp.float32),
            pltpu.SemaphoreType.DMA,
        ],
    )(bacc)
    return out


# speedup vs baseline: 4.1253x; 4.1253x over previous
import jax
import jax.numpy as jnp
from jax import lax
from jax.experimental import pallas as pl
from jax.experimental.pallas import tpu as pltpu
from jax.experimental.pallas import tpu_sc as plsc

BS = 2
C = 96
H = W = 384
NPIX = H * W
ACC_PER_TILE = NPIX // 16
TRASH = NPIX
TRASH_WORDS = 4096
ACC_WORDS = NPIX + TRASH_WORDS
PX_PER_TILE = NPIX // 16
GROUPS = PX_PER_TILE // 16
ROWS_PER_TILE = H // 16
CLIP_HI = 385.0

NCH = 4
GR_PER_CH = GROUPS // NCH
CH_PX = GR_PER_CH * 16


def _sc_body(g_hbm, out_hbm, gv,
             i00, i01, i10, i11, w00, w01, w10, w11,
             zbuf, rbuf, acc, sem):
    c = lax.axis_index("c")
    s = lax.axis_index("s")

    gbase = c * (2 * NPIX) + s * PX_PER_TILE
    ld0 = pltpu.make_async_copy(g_hbm.at[pl.ds(gbase, PX_PER_TILE)],
                                gv.at[pl.ds(0, PX_PER_TILE)], sem)
    ld1 = pltpu.make_async_copy(g_hbm.at[pl.ds(gbase + NPIX, PX_PER_TILE)],
                                gv.at[pl.ds(PX_PER_TILE, PX_PER_TILE)], sem)
    ld0.start()
    ld1.start()

    def _zero(k, _):
        zbuf[pl.ds(k * 16, 16)] = jnp.zeros((16,), jnp.float32)
        return _
    lax.fori_loop(0, ACC_PER_TILE // 16, _zero, None)
    pltpu.sync_copy(zbuf, acc.at[pl.ds(s * ACC_PER_TILE, ACC_PER_TILE)])

    plsc.subcore_barrier()
    ld0.wait()
    ld1.wait()

    lane = lax.iota(jnp.int32, 16)

    def _compute(g, _):
        col = g * 16
        trash = (TRASH + (col & (TRASH_WORDS - 16))) + lane
        gi = gv[pl.ds(g * 16, 16)]
        gj = gv[pl.ds(PX_PER_TILE + g * 16, 16)]
        ti = jnp.minimum(jnp.maximum(gi * 192.0 + 193.0, 0.0), CLIP_HI)
        tj = jnp.minimum(jnp.maximum(gj * 192.0 + 193.0, 0.0), CLIP_HI)
        li = ti.astype(jnp.int32)
        lj = tj.astype(jnp.int32)
        fi = ti - li.astype(jnp.float32)
        fj = tj - lj.astype(jnp.float32)
        base = li * W + lj - (W + 1)
        mi0 = jnp.logical_and(li >= 1, li <= H)
        mi1 = li <= H - 1
        mj0 = jnp.logical_and(lj >= 1, lj <= W)
        mj1 = lj <= W - 1
        i00[pl.ds(col, 16)] = jnp.where(mi0 & mj0, base, trash)
        i01[pl.ds(col, 16)] = jnp.where(mi0 & mj1, base + 1, trash)
        i10[pl.ds(col, 16)] = jnp.where(mi1 & mj0, base + W, trash)
        i11[pl.ds(col, 16)] = jnp.where(mi1 & mj1, base + (W + 1), trash)
        a0 = 1.0 - fi
        b0 = 1.0 - fj
        w00[pl.ds(col, 16)] = a0 * b0
        w01[pl.ds(col, 16)] = a0 * fj
        w10[pl.ds(col, 16)] = fi * b0
        w11[pl.ds(col, 16)] = fi * fj
        return _

    descs = []
    for k in range(NCH):
        lax.fori_loop(k * GR_PER_CH, (k + 1) * GR_PER_CH, _compute, None)
        sl = pl.ds(k * CH_PX, CH_PX)
        for iv, wv in ((i00, w00), (i01, w01), (i10, w10), (i11, w11)):
            descs.append(
                pltpu.async_copy(wv.at[sl], acc.at[iv.at[sl]], sem, add=True))
    for d in descs:
        d.wait()

    plsc.subcore_barrier()

    rdescs = []
    for r in range(ROWS_PER_TILE):
        rdescs.append(pltpu.async_copy(
            acc.at[pl.ds(s * ACC_PER_TILE + r * W, W)], rbuf.at[r], sem))
    for d in rdescs:
        d.wait()
    row0 = pl.multiple_of(c * H + s * ROWS_PER_TILE, ROWS_PER_TILE)
    pltpu.sync_copy(rbuf, out_hbm.at[pl.ds(row0, ROWS_PER_TILE)])


_sc_scatter = pl.kernel(
    _sc_body,
    out_type=jax.ShapeDtypeStruct((BS * H, W), jnp.float32),
    mesh=plsc.VectorSubcoreMesh(core_axis_name="c", subcore_axis_name="s"),
    scratch_types=[
        pltpu.VMEM((2 * PX_PER_TILE,), jnp.float32),
        pltpu.VMEM((PX_PER_TILE,), jnp.int32),
        pltpu.VMEM((PX_PER_TILE,), jnp.int32),
        pltpu.VMEM((PX_PER_TILE,), jnp.int32),
        pltpu.VMEM((PX_PER_TILE,), jnp.int32),
        pltpu.VMEM((PX_PER_TILE,), jnp.float32),
        pltpu.VMEM((PX_PER_TILE,), jnp.float32),
        pltpu.VMEM((PX_PER_TILE,), jnp.float32),
        pltpu.VMEM((PX_PER_TILE,), jnp.float32),
        pltpu.VMEM((ACC_PER_TILE,), jnp.float32),
        pltpu.VMEM((ROWS_PER_TILE, W), jnp.float32),
        pltpu.VMEM_SHARED((ACC_WORDS,), jnp.float32),
        pltpu.SemaphoreType.DMA,
    ],
)


CB = 8


def _bcast_body(b_ref, o_ref, s0, s1, sem):
    descs = []
    for b, s in ((0, s0), (1, s1)):
        s[...] = jnp.broadcast_to(b_ref[pl.ds(b * H, H), :][None], (CB, H, W))
        for c in range(0, C, CB):
            d = pltpu.make_async_copy(s, o_ref.at[b, pl.ds(c, CB)], sem)
            d.start()
            descs.append(d)
    for d in descs:
        d.wait()


def kernel(x, inv_grid):
    g = jnp.transpose(inv_grid, (0, 3, 1, 2)).reshape(BS * 2 * NPIX)
    bacc = _sc_scatter(g)

    out = pl.pallas_call(
        _bcast_body,
        in_specs=[pl.BlockSpec(memory_space=pltpu.VMEM)],
        out_specs=pl.BlockSpec(memory_space=pltpu.MemorySpace.HBM),
        out_shape=jax.ShapeDtypeStruct((BS, C, H, W), x.dtype),
        scratch_shapes=[
            pltpu.VMEM((CB, H, W), jnp.float32),
            pltpu.VMEM((CB, H, W), jnp.float32),
            pltpu.SemaphoreType.DMA,
        ],
    )(bacc)
    return out


# speedup vs baseline: 4.1427x; 1.0042x over previous
import jax
import jax.numpy as jnp
from jax import lax
from jax.experimental import pallas as pl
from jax.experimental.pallas import tpu as pltpu
from jax.experimental.pallas import tpu_sc as plsc

BS = 2
C = 96
H = W = 384
NPIX = H * W
ACC_PER_TILE = NPIX // 16
TRASH = NPIX
TRASH_WORDS = 4096
ACC_WORDS = NPIX + TRASH_WORDS
PX_PER_TILE = NPIX // 16
GROUPS = PX_PER_TILE // 16
ROWS_PER_TILE = H // 16
CLIP_HI = 385.0

CHUNKS = (0, 72, 240, 408, 576)


def _sc_body(g_hbm, out_hbm, gv,
             i00, i01, i10, i11, w00, w01, w10, w11,
             zbuf, rbuf, acc, sem):
    c = lax.axis_index("c")
    s = lax.axis_index("s")

    rA = pl.multiple_of(c * (2 * H) + s * ROWS_PER_TILE, 8)
    rB = pl.multiple_of(c * (2 * H) + H + s * ROWS_PER_TILE, 8)
    ld0 = pltpu.make_async_copy(g_hbm.at[pl.ds(rA, ROWS_PER_TILE)],
                                gv.at[pl.ds(0, ROWS_PER_TILE)], sem)
    ld1 = pltpu.make_async_copy(g_hbm.at[pl.ds(rB, ROWS_PER_TILE)],
                                gv.at[pl.ds(ROWS_PER_TILE, ROWS_PER_TILE)],
                                sem)
    ld0.start()
    ld1.start()

    def _zero(k, _):
        zbuf[pl.ds(k * 16, 16)] = jnp.zeros((16,), jnp.float32)
        return _
    lax.fori_loop(0, ACC_PER_TILE // 16, _zero, None)
    pltpu.sync_copy(zbuf, acc.at[pl.ds(s * ACC_PER_TILE, ACC_PER_TILE)])

    plsc.subcore_barrier()
    ld0.wait()
    ld1.wait()

    lane = lax.iota(jnp.int32, 16)

    def _compute(g, _):
        col = g * 16
        trash = (TRASH + (col & (TRASH_WORDS - 16))) + lane
        gr = g // ROWS_PER_TILE
        gc = (g % ROWS_PER_TILE) * 16
        gi = gv[gr, pl.ds(gc, 16)]
        gj = gv[gr + ROWS_PER_TILE, pl.ds(gc, 16)]
        ti = jnp.minimum(jnp.maximum(gi * 192.0 + 193.0, 0.0), CLIP_HI)
        tj = jnp.minimum(jnp.maximum(gj * 192.0 + 193.0, 0.0), CLIP_HI)
        li = ti.astype(jnp.int32)
        lj = tj.astype(jnp.int32)
        fi = ti - li.astype(jnp.float32)
        fj = tj - lj.astype(jnp.float32)
        base = li * W + lj - (W + 1)
        mi0 = jnp.logical_and(li >= 1, li <= H)
        mi1 = li <= H - 1
        mj0 = jnp.logical_and(lj >= 1, lj <= W)
        mj1 = lj <= W - 1
        i00[pl.ds(col, 16)] = jnp.where(mi0 & mj0, base, trash)
        i01[pl.ds(col, 16)] = jnp.where(mi0 & mj1, base + 1, trash)
        i10[pl.ds(col, 16)] = jnp.where(mi1 & mj0, base + W, trash)
        i11[pl.ds(col, 16)] = jnp.where(mi1 & mj1, base + (W + 1), trash)
        a0 = 1.0 - fi
        b0 = 1.0 - fj
        w00[pl.ds(col, 16)] = a0 * b0
        w01[pl.ds(col, 16)] = a0 * fj
        w10[pl.ds(col, 16)] = fi * b0
        w11[pl.ds(col, 16)] = fi * fj
        return _

    descs = []
    for k in range(len(CHUNKS) - 1):
        lo, hi = CHUNKS[k], CHUNKS[k + 1]
        lax.fori_loop(lo, hi, _compute, None)
        sl = pl.ds(lo * 16, (hi - lo) * 16)
        for iv, wv in ((i00, w00), (i01, w01), (i10, w10), (i11, w11)):
            descs.append(
                pltpu.async_copy(wv.at[sl], acc.at[iv.at[sl]], sem, add=True))
    for d in descs:
        d.wait()

    plsc.subcore_barrier()

    rdescs = []
    for r in range(ROWS_PER_TILE):
        rdescs.append(pltpu.async_copy(
            acc.at[pl.ds(s * ACC_PER_TILE + r * W, W)], rbuf.at[r], sem))
    for d in rdescs:
        d.wait()
    row0 = pl.multiple_of(c * H + s * ROWS_PER_TILE, ROWS_PER_TILE)
    pltpu.sync_copy(rbuf, out_hbm.at[pl.ds(row0, ROWS_PER_TILE)])


_sc_scatter = pl.kernel(
    _sc_body,
    out_type=jax.ShapeDtypeStruct((BS * H, W), jnp.float32),
    mesh=plsc.VectorSubcoreMesh(core_axis_name="c", subcore_axis_name="s"),
    scratch_types=[
        pltpu.VMEM((2 * ROWS_PER_TILE, W), jnp.float32),
        pltpu.VMEM((PX_PER_TILE,), jnp.int32),
        pltpu.VMEM((PX_PER_TILE,), jnp.int32),
        pltpu.VMEM((PX_PER_TILE,), jnp.int32),
        pltpu.VMEM((PX_PER_TILE,), jnp.int32),
        pltpu.VMEM((PX_PER_TILE,), jnp.float32),
        pltpu.VMEM((PX_PER_TILE,), jnp.float32),
        pltpu.VMEM((PX_PER_TILE,), jnp.float32),
        pltpu.VMEM((PX_PER_TILE,), jnp.float32),
        pltpu.VMEM((ACC_PER_TILE,), jnp.float32),
        pltpu.VMEM((ROWS_PER_TILE, W), jnp.float32),
        pltpu.VMEM_SHARED((ACC_WORDS,), jnp.float32),
        pltpu.SemaphoreType.DMA,
    ],
)


CB = 8


def _bcast_body(b_ref, o_ref, s0, s1, sem):
    descs = []
    for b, s in ((0, s0), (1, s1)):
        s[...] = jnp.broadcast_to(b_ref[pl.ds(b * H, H), :][None], (CB, H, W))
        for c in range(0, C, CB):
            d = pltpu.make_async_copy(s, o_ref.at[b, pl.ds(c, CB)], sem)
            d.start()
            descs.append(d)
    for d in descs:
        d.wait()


def kernel(x, inv_grid):
    g = jnp.transpose(inv_grid, (0, 3, 1, 2)).reshape(BS * 2 * H, W)
    bacc = _sc_scatter(g)

    out = pl.pallas_call(
        _bcast_body,
        in_specs=[pl.BlockSpec(memory_space=pltpu.VMEM)],
        out_specs=pl.BlockSpec(memory_space=pltpu.MemorySpace.HBM),
        out_shape=jax.ShapeDtypeStruct((BS, C, H, W), x.dtype),
        scratch_shapes=[
            pltpu.VMEM((CB, H, W), jnp.float32),
            pltpu.VMEM((CB, H, W), jnp.float32),
            pltpu.SemaphoreType.DMA,
        ],
    )(bacc)
    return out
